# baseline (speedup 1.0000x reference)
import functools

import jax
import jax.numpy as jnp
from jax import lax
from jax.experimental import pallas as pl
from jax.experimental.pallas import tpu as pltpu
from jax.experimental.pallas import tpu_sc as plsc

_NC = 2
_NS = 16
_NW = _NC * _NS
_LANES = 16


@functools.cache
def _make_kernel(batch, seq, emb):
    blk = batch // _NW
    assert blk == 128 and emb % 8 == 0 and seq % 2 == 0
    te = emb // 8
    mesh = plsc.VectorSubcoreMesh(core_axis_name="c", subcore_axis_name="s")

    @functools.partial(
        pl.kernel,
        out_type=jax.ShapeDtypeStruct((seq, te, _NW, 8, 128), jnp.float32),
        mesh=mesh,
        compiler_params=pltpu.CompilerParams(use_tc_tiling_on_sc=False,
                                             needs_layout_passes=False),
        scratch_types=[
            pltpu.VMEM((seq, blk), jnp.int32),
            pltpu.VMEM((seq, emb), jnp.float32),
            pltpu.VMEM((blk, 80), jnp.float32),
            pltpu.VMEM((blk, 80), jnp.float32),
            pltpu.VMEM((te, 8, 128), jnp.float32),
            pltpu.VMEM((te, 8, 128), jnp.float32),
            pltpu.SemaphoreType.DMA,
            pltpu.SemaphoreType.DMA,
            pltpu.SemaphoreType.DMA,
            pltpu.SemaphoreType.DMA,
        ],
    )
    def emb_kernel(idx_hbm, tok_hbm, pos_hbm, out_hbm,
                   idx_all, pos_v, rows0, rows1, st0, st1,
                   g0, g1, o0, o1):
        wid = lax.axis_index("s") * _NC + lax.axis_index("c")
        pltpu.sync_copy(idx_hbm.at[:, wid], idx_all)
        pltpu.sync_copy(pos_hbm, pos_v)

        rows = (rows0, rows1)
        staged = (st0, st1)
        gsems = (g0, g1)
        osems = (o0, o1)
        lane = lax.iota(jnp.int32, _LANES)

        def gather_cp(s, buf):
            return (tok_hbm.at[idx_all.at[s]], rows[buf], gsems[buf])

        def out_cp(s, buf):
            return (staged[buf], out_hbm.at[s, :, wid], osems[buf])

        def start(args):
            pltpu.async_copy(*args)

        def wait(args):
            pltpu.make_async_copy(*args).wait()

        rids = [blk16 * _LANES + lane for blk16 in range(blk // _LANES)]

        def process(s, buf):
            rv = rows[buf]
            sv = staged[buf]
            srow = jnp.full((_LANES,), s, jnp.int32)

            def body(e, c):
                col = jnp.full((_LANES,), e, jnp.int32)
                pv = plsc.load_gather(pos_v, [srow, col])
                for blk16 in range(blk // _LANES):
                    vals = plsc.load_gather(rv, [rids[blk16], col]) + pv
                    sv[e // 8, e % 8, pl.ds(blk16 * _LANES, _LANES)] = vals
                return c

            lax.fori_loop(0, emb, body, 0)

        start(gather_cp(0, 0))

        def pair(k, c):
            s = 2 * k
            start(gather_cp(s + 1, 1))
            wait(gather_cp(s, 0))

            @pl.when(k > 0)
            def _():
                wait(out_cp(s - 2, 0))

            process(s, 0)
            start(out_cp(s, 0))

            @pl.when(k < seq // 2 - 1)
            def _():
                start(gather_cp(s + 2, 0))

            wait(gather_cp(s + 1, 1))

            @pl.when(k > 0)
            def _():
                wait(out_cp(s - 1, 1))

            process(s + 1, 1)
            start(out_cp(s + 1, 1))
            return c

        lax.fori_loop(0, seq // 2, pair, 0)
        wait(out_cp(seq - 2, 0))
        wait(out_cp(seq - 1, 1))

    return emb_kernel


def kernel(inputs, token_table, position_table):
    batch, seq = inputs.shape
    emb = token_table.shape[1]
    idx_t = inputs.astype(jnp.int32).T.reshape(seq, _NW, batch // _NW)
    tok80 = jnp.pad(token_table, ((0, 0), (0, 80 - emb)))
    f = _make_kernel(batch, seq, emb)
    out = f(idx_t, tok80, position_table)
    out = out.transpose(2, 4, 0, 1, 3).reshape(batch, seq, emb)
    return out


# speedup vs baseline: 1.2333x; 1.2333x over previous
import functools

import jax
import jax.numpy as jnp
from jax import lax
from jax.experimental import pallas as pl
from jax.experimental.pallas import tpu as pltpu
from jax.experimental.pallas import tpu_sc as plsc

_NC = 2
_NS = 16
_NW = _NC * _NS
_LANES = 16


@functools.cache
def _make_kernel(batch, seq, emb):
    rows_per_w = batch // _NW
    assert rows_per_w % 2 == 0 and rows_per_w >= 6
    chunk_a = 104
    chunk_b = seq - chunk_a
    mesh = plsc.VectorSubcoreMesh(core_axis_name="c", subcore_axis_name="s")

    @functools.partial(
        pl.kernel,
        out_type=jax.ShapeDtypeStruct((batch, seq, emb), jnp.float32),
        mesh=mesh,
        compiler_params=pltpu.CompilerParams(use_tc_tiling_on_sc=True),
        scratch_types=[
            pltpu.VMEM((seq * emb,), jnp.float32),
            pltpu.VMEM((seq,), jnp.int32),
            pltpu.VMEM((seq,), jnp.int32),
            pltpu.VMEM((seq, 128), jnp.float32),
            pltpu.VMEM((seq, 128), jnp.float32),
            pltpu.VMEM((seq, emb), jnp.float32),
            pltpu.VMEM((seq, emb), jnp.float32),
            pltpu.SemaphoreType.DMA,
            pltpu.SemaphoreType.DMA,
            pltpu.SemaphoreType.DMA,
            pltpu.SemaphoreType.DMA,
            pltpu.SemaphoreType.DMA,
            pltpu.SemaphoreType.DMA,
        ],
    )
    def emb_kernel(idx_hbm, tok_hbm, pos_hbm, out_hbm,
                   pos_v, idx0, idx1, rows0, rows1, st0, st1,
                   is0, is1, in0, in1, os0, os1):
        wid = lax.axis_index("s") * _NC + lax.axis_index("c")
        base = wid * rows_per_w
        pltpu.sync_copy(pos_hbm, pos_v)

        idxs = (idx0, idx1)
        rows = (rows0, rows1)
        staged = (st0, st1)
        isems = (is0, is1)
        gsems = (in0, in1)
        osems = (os0, os1)

        def idx_cp(r_local, buf):
            return (idx_hbm.at[pl.ds((base + r_local) * seq, seq)],
                    idxs[buf], isems[buf])

        def gather_cps(buf):
            return [
                (tok_hbm.at[idxs[buf].at[pl.ds(0, chunk_a)]],
                 rows[buf].at[pl.ds(0, chunk_a)],
                 gsems[buf]),
                (tok_hbm.at[idxs[buf].at[pl.ds(chunk_a, chunk_b)]],
                 rows[buf].at[pl.ds(chunk_a, chunk_b)],
                 gsems[buf]),
            ]

        def out_cp(r_local, buf):
            return (staged[buf], out_hbm.at[base + r_local], osems[buf])

        def start(args):
            pltpu.async_copy(*args)

        def wait(args):
            pltpu.make_async_copy(*args).wait()

        def add_pos(buf):
            rv = rows[buf]
            sv = staged[buf]

            def body(i, c):
                for k in range(emb // _LANES):
                    sl = pl.ds(k * _LANES, _LANES)
                    sv[i, sl] = rv[i, sl] + pos_v[pl.ds(i * emb + k * _LANES,
                                                        _LANES)]
                return c

            lax.fori_loop(0, seq, body, 0)

        def iteration(r, b, *, warm_out, feed_gather, feed_idx):
            b2 = 1 - b
            if feed_gather:
                wait(idx_cp(r + 1, b2))
                for args in gather_cps(b2):
                    start(args)
            for args in gather_cps(b):
                wait(args)
            if feed_idx:
                start(idx_cp(r + 2, b))
            if warm_out:
                wait(out_cp(r - 2, b))
            add_pos(b)
            start(out_cp(r, b))

        start(idx_cp(0, 0))
        start(idx_cp(1, 1))
        wait(idx_cp(0, 0))
        for args in gather_cps(0):
            start(args)

        iteration(0, 0, warm_out=False, feed_gather=True, feed_idx=True)
        iteration(1, 1, warm_out=False, feed_gather=True, feed_idx=True)

        def pair(k, c):
            r = 2 * k + 2
            iteration(r, 0, warm_out=True, feed_gather=True, feed_idx=True)
            iteration(r + 1, 1, warm_out=True, feed_gather=True, feed_idx=True)
            return c

        lax.fori_loop(0, (rows_per_w - 4) // 2, pair, 0)

        iteration(rows_per_w - 2, 0, warm_out=True, feed_gather=True,
                  feed_idx=False)
        iteration(rows_per_w - 1, 1, warm_out=True, feed_gather=False,
                  feed_idx=False)
        wait(out_cp(rows_per_w - 2, 0))
        wait(out_cp(rows_per_w - 1, 1))

    return emb_kernel


def kernel(inputs, token_table, position_table):
    batch, seq = inputs.shape
    emb = token_table.shape[1]
    idx = inputs.astype(jnp.int32).reshape(batch * seq)
    tok128 = jnp.pad(token_table, ((0, 0), (0, 128 - emb)))
    pos_flat = position_table.reshape(seq * emb)
    f = _make_kernel(batch, seq, emb)
    return f(idx, tok128, pos_flat)
